# baseline (speedup 1.0000x reference)
import functools

import jax
import jax.numpy as jnp
from jax import lax
from jax.experimental import pallas as pl
from jax.experimental.pallas import tpu as pltpu
from jax.experimental.pallas import tpu_sc as plsc

VOCAB = 1000000
EMBED_DIM = 32
BATCH = 4096
HIST = 200

NUM_CORES = 2
NUM_SUBCORES = 16
NUM_WORKERS = NUM_CORES * NUM_SUBCORES

NROW = 128
UNITS_PER_CHUNK = 4
CROWS = NROW * UNITS_PER_CHUNK
TOTAL = BATCH * HIST
NCHUNK = TOTAL // CROWS
PER_W = NCHUNK // NUM_WORKERS
NBUF = 2
NITER = PER_W // NBUF

HB = HIST // 8
BB = BATCH // NROW
JB = EMBED_DIM // 8

_MESH = plsc.VectorSubcoreMesh(
    core_axis_name="c", subcore_axis_name="s",
    num_cores=NUM_CORES, num_subcores=NUM_SUBCORES,
)


@functools.partial(
    pl.kernel,
    out_type=jax.ShapeDtypeStruct((HIST, JB, BB, 8, NROW), jnp.float32),
    mesh=_MESH,
    scratch_types=(
        [pltpu.VMEM((PER_W * CROWS,), jnp.int32)]
        + [pltpu.VMEM((CROWS, EMBED_DIM), jnp.float32)] * NBUF
        + [pltpu.VMEM((UNITS_PER_CHUNK, JB, 8, NROW), jnp.float32)] * NBUF
        + [pltpu.SemaphoreType.DMA] * (2 * NBUF)
    ),
    compiler_params=pltpu.CompilerParams(
        use_tc_tiling_on_sc=False, needs_layout_passes=False),
)
def _gather_kernel(idx_hbm, table_hbm, out_hbm, idxblk,
                   g0, g1, t0, t1, gs0, gs1, ws0, ws1):
    gbuf = (g0, g1)
    tbuf = (t0, t1)
    gsem = (gs0, gs1)
    wsem = (ws0, ws1)

    wid = lax.axis_index("s") * NUM_CORES + lax.axis_index("c")
    cbase = wid * PER_W

    pltpu.sync_copy(idx_hbm.at[pl.ds(cbase * CROWS, PER_W * CROWS)], idxblk)

    iota16 = lax.iota(jnp.int32, 16)

    def g_start(lc, b):
        return pltpu.async_copy(
            table_hbm.at[idxblk.at[pl.ds(lc * CROWS, CROWS)]], gbuf[b], gsem[b])

    def g_wait(b):
        pltpu.make_async_copy(
            table_hbm.at[idxblk.at[pl.ds(0, CROWS)]], gbuf[b], gsem[b]).wait()

    def w_start(lc, b):
        v0 = (cbase + lc) * UNITS_PER_CHUNK
        h0 = 8 * (v0 // (BB * 8)) + v0 % 8
        bc = (v0 // 8) % BB
        return pltpu.async_copy(
            tbuf[b], out_hbm.at[pl.ds(h0, UNITS_PER_CHUNK), :, bc], wsem[b])

    def w_wait(b):
        pltpu.make_async_copy(
            tbuf[b], out_hbm.at[pl.ds(0, UNITS_PER_CHUNK), :, 0], wsem[b]).wait()

    def transpose(b):
        def u4body(u4, carry):
            cvs = [iota16 + (NROW * u4 + 16 * cb) for cb in range(8)]
            for j in range(EMBED_DIM):
                jvec = jnp.full((16,), j, jnp.int32)
                for cb in range(8):
                    vals = plsc.load_gather(gbuf[b], [cvs[cb], jvec])
                    tbuf[b][u4, j // 8, j % 8, pl.ds(16 * cb, 16)] = vals
            return carry
        lax.fori_loop(0, UNITS_PER_CHUNK, u4body, 0)

    for b in range(NBUF):
        g_start(b, b)
    for b in range(NBUF):
        g_wait(b)
        transpose(b)
        w_start(b, b)
        g_start(b + NBUF, b)

    def round_body(i, carry):
        for b in range(NBUF):
            lc = NBUF * i + b
            g_wait(b)
            w_wait(b)
            transpose(b)
            w_start(lc, b)
            g_start(jnp.minimum(lc + NBUF, PER_W - 1), b)
        return carry

    lax.fori_loop(1, NITER, round_body, 0)

    for b in range(NBUF):
        g_wait(b)
    for b in range(NBUF):
        w_wait(b)


def kernel(wordtypes, table):
    idx = (wordtypes.T.reshape(HB, 8, BB, NROW)
           .transpose(0, 2, 1, 3)
           .reshape(TOTAL))
    out5 = _gather_kernel(idx, table)
    out = (out5.transpose(0, 1, 3, 2, 4)
           .reshape(HIST, EMBED_DIM, BATCH)
           .transpose(2, 0, 1))
    return out
